# baseline (speedup 1.0000x reference)
import functools

import jax
import jax.numpy as jnp
from jax import lax
from jax.experimental import pallas as pl
from jax.experimental.pallas import tpu as pltpu
from jax.experimental.pallas import tpu_sc as plsc

_NCLS = 8
_NC = 2
_NS = 16
_NW = _NC * _NS
_L = 16
_C = 512


def _make_sc_kernel(n, d):
    cols_per_w = n // _NW
    nchunks = cols_per_w // _C
    mesh = plsc.VectorSubcoreMesh(core_axis_name="c", subcore_axis_name="s")

    @functools.partial(
        pl.kernel,
        out_type=jax.ShapeDtypeStruct((d, n), jnp.float32),
        mesh=mesh,
        scratch_types=[
            pltpu.VMEM((_NCLS * d,), jnp.float32),
            pltpu.VMEM((_NCLS * d,), jnp.float32),
            pltpu.VMEM((d, _C), jnp.float32),
            pltpu.VMEM((d, _C), jnp.float32),
            pltpu.VMEM((d, _C), jnp.float32),
            pltpu.VMEM((d, _C), jnp.float32),
            pltpu.VMEM((_C,), jnp.int32),
            pltpu.VMEM((_C,), jnp.int32),
            pltpu.SemaphoreType.DMA,
            pltpu.SemaphoreType.DMA,
            pltpu.SemaphoreType.DMA,
            pltpu.SemaphoreType.DMA,
        ],
        compiler_params=pltpu.CompilerParams(
            needs_layout_passes=False, use_tc_tiling_on_sc=False),
    )
    def sc_kernel(x_hbm, lab_hbm, scale_hbm, shift_hbm, out_hbm,
                  sc_v, sh_v, xb0, xb1, ob0, ob1, lb0, lb1,
                  lsem0, lsem1, ssem0, ssem1):
        xb = (xb0, xb1)
        ob = (ob0, ob1)
        lb = (lb0, lb1)
        lsem = (lsem0, lsem1)
        ssem = (ssem0, ssem1)

        wid = lax.axis_index("s") * _NC + lax.axis_index("c")
        base = wid * cols_per_w

        pltpu.sync_copy(scale_hbm, sc_v)
        pltpu.sync_copy(shift_hbm, sh_v)

        def start_load(col0, b):
            pltpu.make_async_copy(
                x_hbm.at[:, pl.ds(col0, _C)], xb[b], lsem[b]).start()
            pltpu.make_async_copy(
                lab_hbm.at[pl.ds(col0, _C)], lb[b], lsem[b]).start()

        def wait_load(b):
            pltpu.make_async_copy(
                x_hbm.at[:, pl.ds(0, _C)], xb[b], lsem[b]).wait()
            pltpu.make_async_copy(
                lab_hbm.at[pl.ds(0, _C)], lb[b], lsem[b]).wait()

        def start_store(col0, b):
            pltpu.make_async_copy(
                ob[b], out_hbm.at[:, pl.ds(col0, _C)], ssem[b]).start()

        def wait_store(b):
            pltpu.make_async_copy(
                ob[b], out_hbm.at[:, pl.ds(0, _C)], ssem[b]).wait()

        def compute(b):
            def group(g, carry):
                c16 = pl.ds(g * _L, _L)
                lv = lb[b][c16]
                for dd in range(d):
                    idx = lv + dd * _NCLS
                    s = plsc.load_gather(sc_v, [idx])
                    t = plsc.load_gather(sh_v, [idx])
                    ob[b][dd, c16] = xb[b][dd, c16] * s + t
                return carry
            lax.fori_loop(0, _C // _L, group, 0)

        start_load(base, 0)

        def outer(c2, carry):
            for b in range(2):
                cc = c2 * 2 + b
                nxt = jnp.minimum(base + (cc + 1) * _C, n - _C)
                start_load(nxt, 1 - b)
                wait_load(b)
                @pl.when(cc >= 2)
                def _():
                    wait_store(b)
                compute(b)
                start_store(base + cc * _C, b)
            return carry

        lax.fori_loop(0, nchunks // 2, outer, 0)

        wait_load(0)
        wait_store(0)
        wait_store(1)

    return sc_kernel


def kernel(x, labels, scale, shift):
    n, d = x.shape
    xt = jnp.swapaxes(x, 0, 1)
    sck = _make_sc_kernel(n, d)
    st = jnp.swapaxes(scale, 0, 1).reshape(-1)
    tt = jnp.swapaxes(shift, 0, 1).reshape(-1)
    out_t = sck(xt, labels.astype(jnp.int32), st, tt)
    return jnp.swapaxes(out_t, 0, 1)


# speedup vs baseline: 8.5460x; 8.5460x over previous
import functools

import jax
import jax.numpy as jnp
from jax import lax
from jax.experimental import pallas as pl
from jax.experimental.pallas import tpu as pltpu
from jax.experimental.pallas import tpu_sc as plsc

_NCLS = 8
_NC = 2
_NS = 16
_NW = _NC * _NS
_L = 16
_DT = 4
_DS = 8
_LN = 128
_CM = 4


def _make_sc_kernel(n, d):
    ntiles = n // _LN
    tiles_per_w = ntiles // _NW
    nchunks = tiles_per_w // _CM
    mesh = plsc.VectorSubcoreMesh(core_axis_name="c", subcore_axis_name="s")

    @functools.partial(
        pl.kernel,
        out_type=jax.ShapeDtypeStruct((_DT, ntiles, _DS, _LN), jnp.float32),
        mesh=mesh,
        scratch_types=[
            pltpu.VMEM((_NCLS * d,), jnp.float32),
            pltpu.VMEM((_NCLS * d,), jnp.float32),
            pltpu.VMEM((_DT, _CM, _DS, _LN), jnp.float32),
            pltpu.VMEM((_DT, _CM, _DS, _LN), jnp.float32),
            pltpu.VMEM((_DT, _CM, _DS, _LN), jnp.float32),
            pltpu.VMEM((_DT, _CM, _DS, _LN), jnp.float32),
            pltpu.VMEM((_CM * _LN,), jnp.int32),
            pltpu.VMEM((_CM * _LN,), jnp.int32),
            pltpu.SemaphoreType.DMA,
            pltpu.SemaphoreType.DMA,
            pltpu.SemaphoreType.DMA,
            pltpu.SemaphoreType.DMA,
        ],
        compiler_params=pltpu.CompilerParams(
            needs_layout_passes=False, use_tc_tiling_on_sc=False),
    )
    def sc_kernel(x_hbm, lab_hbm, scale_hbm, shift_hbm, out_hbm,
                  sc_v, sh_v, xb0, xb1, ob0, ob1, lb0, lb1,
                  lsem0, lsem1, ssem0, ssem1):
        xb = (xb0, xb1)
        ob = (ob0, ob1)
        lb = (lb0, lb1)
        lsem = (lsem0, lsem1)
        ssem = (ssem0, ssem1)

        wid = lax.axis_index("s") * _NC + lax.axis_index("c")
        base = wid * tiles_per_w

        pltpu.sync_copy(scale_hbm, sc_v)
        pltpu.sync_copy(shift_hbm, sh_v)

        def start_load(m0, b):
            pltpu.make_async_copy(
                x_hbm.at[:, pl.ds(m0, _CM), :, :], xb[b], lsem[b]).start()
            pltpu.make_async_copy(
                lab_hbm.at[pl.ds(m0 * _LN, _CM * _LN)], lb[b], lsem[b]).start()

        def wait_load(b):
            pltpu.make_async_copy(
                x_hbm.at[:, pl.ds(0, _CM), :, :], xb[b], lsem[b]).wait()
            pltpu.make_async_copy(
                lab_hbm.at[pl.ds(0, _CM * _LN)], lb[b], lsem[b]).wait()

        def start_store(m0, b):
            pltpu.make_async_copy(
                ob[b], out_hbm.at[:, pl.ds(m0, _CM), :, :], ssem[b]).start()

        def wait_store(b):
            pltpu.make_async_copy(
                ob[b], out_hbm.at[:, pl.ds(0, _CM), :, :], ssem[b]).wait()

        def compute(b):
            def group(lg, carry):
                l16 = pl.ds(lg * _L, _L)
                for m in range(_CM):
                    lv = lb[b][pl.ds(m * _LN + lg * _L, _L)]
                    for t in range(_DT):
                        for s in range(_DS):
                            idx = lv + (t * _DS + s) * _NCLS
                            sv = plsc.load_gather(sc_v, [idx])
                            tv = plsc.load_gather(sh_v, [idx])
                            xv = xb[b][t, m, s, l16]
                            ob[b][t, m, s, l16] = xv * sv + tv
                return carry
            lax.fori_loop(0, _LN // _L, group, 0)

        start_load(base, 0)

        def outer(c2, carry):
            for b in range(2):
                cc = c2 * 2 + b
                nxt = jnp.minimum(base + (cc + 1) * _CM, ntiles - _CM)
                start_load(nxt, 1 - b)
                wait_load(b)
                @pl.when(cc >= 2)
                def _():
                    wait_store(b)
                compute(b)
                start_store(base + cc * _CM, b)
            return carry

        lax.fori_loop(0, nchunks // 2, outer, 0)

        wait_load(0)
        wait_store(0)
        wait_store(1)

    return sc_kernel


def kernel(x, labels, scale, shift):
    n, d = x.shape
    x4 = x.reshape(n // _LN, _LN, _DT, _DS).transpose(2, 0, 3, 1)
    sck = _make_sc_kernel(n, d)
    st = jnp.swapaxes(scale, 0, 1).reshape(-1)
    tt = jnp.swapaxes(shift, 0, 1).reshape(-1)
    out4 = sck(x4, labels.astype(jnp.int32), st, tt)
    return out4.transpose(1, 3, 0, 2).reshape(n, d)


# speedup vs baseline: 25.4035x; 2.9726x over previous
import functools

import jax
import jax.numpy as jnp
from jax import lax
from jax.experimental import pallas as pl
from jax.experimental.pallas import tpu as pltpu
from jax.experimental.pallas import tpu_sc as plsc

_NCLS = 8
_NC = 2
_NS = 16
_NW = _NC * _NS
_L = 16
_DT = 4
_DS = 8
_LN = 128
_CM = 4
_M2 = 1792
_BLK = 16384


def _tc_body(lab_ref, scale_ref, shift_ref, x_ref, o_ref):
    lab = lab_ref[...]
    iot = lax.broadcasted_iota(jnp.int32, (_NCLS, 1), 0)
    oh = (lab == iot).astype(jnp.float32)
    rs = jnp.dot(scale_ref[...], oh, preferred_element_type=jnp.float32)
    rb = jnp.dot(shift_ref[...], oh, preferred_element_type=jnp.float32)
    o_ref[...] = x_ref[...] * rs + rb


def _tc_call(xt, lab2, st, tt, n, d, n_tc):
    grid = (n_tc // _BLK,)
    return pl.pallas_call(
        _tc_body,
        grid=grid,
        in_specs=[
            pl.BlockSpec((1, _BLK), lambda i: (0, i)),
            pl.BlockSpec((d, _NCLS), lambda i: (0, 0)),
            pl.BlockSpec((d, _NCLS), lambda i: (0, 0)),
            pl.BlockSpec((d, _BLK), lambda i: (0, i)),
        ],
        out_specs=pl.BlockSpec((d, _BLK), lambda i: (0, i)),
        out_shape=jax.ShapeDtypeStruct((d, n), jnp.float32),
        compiler_params=pltpu.CompilerParams(
            dimension_semantics=("arbitrary",),
        ),
    )(lab2, st, tt, xt)


def _make_sc_kernel(n, d):
    ntiles = n // _LN
    tile_lo = ntiles - _M2
    tiles_per_w = _M2 // _NW
    nchunks = tiles_per_w // _CM
    mesh = plsc.VectorSubcoreMesh(core_axis_name="c", subcore_axis_name="s")

    @functools.partial(
        pl.kernel,
        out_type=jax.ShapeDtypeStruct((_DT, _M2, _DS, _LN), jnp.float32),
        mesh=mesh,
        scratch_types=[
            pltpu.VMEM((_NCLS * d,), jnp.float32),
            pltpu.VMEM((_NCLS * d,), jnp.float32),
            pltpu.VMEM((_DT, _CM, _DS, _LN), jnp.float32),
            pltpu.VMEM((_DT, _CM, _DS, _LN), jnp.float32),
            pltpu.VMEM((_DT, _CM, _DS, _LN), jnp.float32),
            pltpu.VMEM((_DT, _CM, _DS, _LN), jnp.float32),
            pltpu.VMEM((_CM * _LN,), jnp.int32),
            pltpu.VMEM((_CM * _LN,), jnp.int32),
            pltpu.SemaphoreType.DMA,
            pltpu.SemaphoreType.DMA,
            pltpu.SemaphoreType.DMA,
            pltpu.SemaphoreType.DMA,
        ],
        compiler_params=pltpu.CompilerParams(
            needs_layout_passes=False, use_tc_tiling_on_sc=False),
    )
    def sc_kernel(x_hbm, lab_hbm, scale_hbm, shift_hbm, out_hbm,
                  sc_v, sh_v, xb0, xb1, ob0, ob1, lb0, lb1,
                  lsem0, lsem1, ssem0, ssem1):
        xb = (xb0, xb1)
        ob = (ob0, ob1)
        lb = (lb0, lb1)
        lsem = (lsem0, lsem1)
        ssem = (ssem0, ssem1)

        wid = lax.axis_index("s") * _NC + lax.axis_index("c")
        base = tile_lo + wid * tiles_per_w

        pltpu.sync_copy(scale_hbm, sc_v)
        pltpu.sync_copy(shift_hbm, sh_v)

        def start_load(m0, b):
            pltpu.make_async_copy(
                x_hbm.at[:, pl.ds(m0, _CM), :, :], xb[b], lsem[b]).start()
            pltpu.make_async_copy(
                lab_hbm.at[pl.ds(m0 * _LN, _CM * _LN)], lb[b], lsem[b]).start()

        def wait_load(b):
            pltpu.make_async_copy(
                x_hbm.at[:, pl.ds(0, _CM), :, :], xb[b], lsem[b]).wait()
            pltpu.make_async_copy(
                lab_hbm.at[pl.ds(0, _CM * _LN)], lb[b], lsem[b]).wait()

        def start_store(m0, b):
            pltpu.make_async_copy(
                ob[b], out_hbm.at[:, pl.ds(m0, _CM), :, :], ssem[b]).start()

        def wait_store(b):
            pltpu.make_async_copy(
                ob[b], out_hbm.at[:, pl.ds(0, _CM), :, :], ssem[b]).wait()

        def compute(b):
            def group(lg, carry):
                l16 = pl.ds(lg * _L, _L)
                for m in range(_CM):
                    lv = lb[b][pl.ds(m * _LN + lg * _L, _L)]
                    for t in range(_DT):
                        for s in range(_DS):
                            idx = lv + (t * _DS + s) * _NCLS
                            sv = plsc.load_gather(sc_v, [idx])
                            tv = plsc.load_gather(sh_v, [idx])
                            xv = xb[b][t, m, s, l16]
                            ob[b][t, m, s, l16] = xv * sv + tv
                return carry
            lax.fori_loop(0, _LN // _L, group, 0)

        start_load(base, 0)

        def outer(c2, carry):
            for b in range(2):
                cc = c2 * 2 + b
                nxt = jnp.minimum(base + (cc + 1) * _CM, ntiles - _CM)
                start_load(nxt, 1 - b)
                wait_load(b)
                @pl.when(cc >= 2)
                def _():
                    wait_store(b)
                compute(b)
                start_store(base - tile_lo + cc * _CM, b)
            return carry

        lax.fori_loop(0, nchunks // 2, outer, 0)

        wait_load(0)
        wait_store(0)
        wait_store(1)

    return sc_kernel


def kernel(x, labels, scale, shift):
    n, d = x.shape
    ntiles = n // _LN
    n_sc = _M2 * _LN
    n_tc = n - n_sc
    labels = labels.astype(jnp.int32)

    xt = jnp.swapaxes(x, 0, 1)
    x4 = x.reshape(ntiles, _LN, _DT, _DS).transpose(2, 0, 3, 1)
    lab2 = labels.reshape(1, n)
    st_tc = jnp.swapaxes(scale, 0, 1)
    tt_tc = jnp.swapaxes(shift, 0, 1)
    st_sc = st_tc.reshape(-1)
    tt_sc = tt_tc.reshape(-1)

    sck = _make_sc_kernel(n, d)
    sc_out4 = sck(x4, labels, st_sc, tt_sc)
    tc_out_t = _tc_call(xt, lab2, st_tc, tt_tc, n, d, n_tc)

    tc_out = jnp.swapaxes(tc_out_t, 0, 1)
    sc_part = sc_out4.transpose(1, 3, 0, 2).reshape(n_sc, d)
    return lax.dynamic_update_slice(tc_out, sc_part, (n_tc, 0))


# speedup vs baseline: 37.9125x; 1.4924x over previous
import jax
import jax.numpy as jnp
from jax import lax
from jax.experimental import pallas as pl
from jax.experimental.pallas import tpu as pltpu

_NCLS = 8
_BLK = 32768


def _body(lab_ref, scale_ref, shift_ref, x_ref, o_ref):
    lab = lab_ref[...]
    iot = lax.broadcasted_iota(jnp.int32, (_NCLS, 1), 0)
    oh = (lab == iot).astype(jnp.float32)
    rs = jnp.dot(scale_ref[...], oh, preferred_element_type=jnp.float32)
    rb = jnp.dot(shift_ref[...], oh, preferred_element_type=jnp.float32)
    o_ref[...] = x_ref[...] * rs + rb


def kernel(x, labels, scale, shift):
    n, d = x.shape
    xt = jnp.swapaxes(x, 0, 1)
    lab2 = labels.reshape(1, n)
    st = jnp.swapaxes(scale, 0, 1)
    tt = jnp.swapaxes(shift, 0, 1)
    grid = (n // _BLK,)
    out_t = pl.pallas_call(
        _body,
        grid=grid,
        in_specs=[
            pl.BlockSpec((1, _BLK), lambda i: (0, i)),
            pl.BlockSpec((d, _NCLS), lambda i: (0, 0)),
            pl.BlockSpec((d, _NCLS), lambda i: (0, 0)),
            pl.BlockSpec((d, _BLK), lambda i: (0, i)),
        ],
        out_specs=pl.BlockSpec((d, _BLK), lambda i: (0, i)),
        out_shape=jax.ShapeDtypeStruct((d, n), x.dtype),
        compiler_params=pltpu.CompilerParams(
            dimension_semantics=("arbitrary",),
        ),
    )(lab2, st, tt, xt)
    return jnp.swapaxes(out_t, 0, 1)


# speedup vs baseline: 38.6984x; 1.0207x over previous
import jax
import jax.numpy as jnp
from jax import lax
from jax.experimental import pallas as pl
from jax.experimental.pallas import tpu as pltpu

_NCLS = 8
_BLK = 65536


def _body(lab_ref, scale_ref, shift_ref, x_ref, o_ref):
    lab = lab_ref[...]
    iot = lax.broadcasted_iota(jnp.int32, (_NCLS, 1), 0)
    oh = (lab == iot).astype(jnp.float32)
    rs = jnp.dot(scale_ref[...], oh, preferred_element_type=jnp.float32)
    rb = jnp.dot(shift_ref[...], oh, preferred_element_type=jnp.float32)
    o_ref[...] = x_ref[...] * rs + rb


def kernel(x, labels, scale, shift):
    n, d = x.shape
    xt = jnp.swapaxes(x, 0, 1)
    lab2 = labels.reshape(1, n)
    st = jnp.swapaxes(scale, 0, 1)
    tt = jnp.swapaxes(shift, 0, 1)
    grid = (n // _BLK,)
    out_t = pl.pallas_call(
        _body,
        grid=grid,
        in_specs=[
            pl.BlockSpec((1, _BLK), lambda i: (0, i)),
            pl.BlockSpec((d, _NCLS), lambda i: (0, 0)),
            pl.BlockSpec((d, _NCLS), lambda i: (0, 0)),
            pl.BlockSpec((d, _BLK), lambda i: (0, i)),
        ],
        out_specs=pl.BlockSpec((d, _BLK), lambda i: (0, i)),
        out_shape=jax.ShapeDtypeStruct((d, n), x.dtype),
        compiler_params=pltpu.CompilerParams(
            dimension_semantics=("arbitrary",),
        ),
    )(lab2, st, tt, xt)
    return jnp.swapaxes(out_t, 0, 1)
